# baseline (speedup 1.0000x reference)
import jax
import jax.numpy as jnp
from jax import lax
from jax.experimental import pallas as pl
from jax.experimental.pallas import tpu as pltpu
from jax.experimental.pallas import tpu_sc as plsc

_ALPHA = 0.1

_NC, _NS, _L = 2, 16, 16
_NW = _NC * _NS
_T = 16384 * 4096
_C = 16384
_E = _T // _NW
_CHUNKS = _E // _C


def _sc_body(x_hbm, o_hbm, in_v, out_v):
    wid = lax.axis_index("s") * _NC + lax.axis_index("c")
    base = wid * _E

    def chunk(k, carry):
        off = base + k * _C
        pltpu.sync_copy(x_hbm.at[pl.ds(off, _C)], in_v)

        def inner(i, c):
            v = in_v[pl.ds(i * _L, _L)]
            out_v[pl.ds(i * _L, _L)] = v + jnp.where(v > -_ALPHA, -_ALPHA, _ALPHA)
            return c

        lax.fori_loop(0, _C // _L, inner, 0)
        pltpu.sync_copy(out_v, o_hbm.at[pl.ds(off, _C)])
        return carry

    lax.fori_loop(0, _CHUNKS, chunk, 0)


def kernel(X):
    M, N = X.shape
    x = X.reshape(-1)
    mesh = plsc.VectorSubcoreMesh(core_axis_name="c", subcore_axis_name="s")
    out = pl.kernel(
        _sc_body,
        mesh=mesh,
        out_type=jax.ShapeDtypeStruct((_T,), jnp.float32),
        scratch_types=[
            pltpu.VMEM((_C,), jnp.float32),
            pltpu.VMEM((_C,), jnp.float32),
        ],
    )(x)
    return out.reshape(M, N)


# speedup vs baseline: 1.9327x; 1.9327x over previous
import jax
import jax.numpy as jnp
from jax import lax
from jax.experimental import pallas as pl
from jax.experimental.pallas import tpu as pltpu
from jax.experimental.pallas import tpu_sc as plsc

_ALPHA = 0.1

_NC, _NS, _L = 2, 16, 16
_NW = _NC * _NS
_T = 16384 * 4096
_C = 16384
_E = _T // _NW
_CHUNKS = _E // _C


def _sc_body(x_hbm, o_hbm, in0, in1, out0, out1, si0, si1, so0, so1):
    wid = lax.axis_index("s") * _NC + lax.axis_index("c")
    base = wid * _E
    bufs = ((in0, out0, si0, so0), (in1, out1, si1, so1))

    for b in range(2):
        pltpu.async_copy(x_hbm.at[pl.ds(base + b * _C, _C)], bufs[b][0], bufs[b][2])

    @pl.loop(0, _CHUNKS, step=2)
    def _outer(g):
        for b in range(2):
            in_v, out_v, in_s, out_s = bufs[b]
            k = g + b
            off = base + k * _C
            pltpu.make_async_copy(x_hbm.at[pl.ds(off, _C)], in_v, in_s).wait()
            @pl.when(k >= 2)
            def _():
                pltpu.make_async_copy(out_v, o_hbm.at[pl.ds(off, _C)], out_s).wait()

            @plsc.parallel_loop(0, _C, step=_L, unroll=8)
            def _compute(i):
                v = in_v[pl.ds(i, _L)]
                out_v[pl.ds(i, _L)] = v + jnp.where(v > -_ALPHA, -_ALPHA, _ALPHA)

            pltpu.async_copy(out_v, o_hbm.at[pl.ds(off, _C)], out_s)

            @pl.when(k + 2 < _CHUNKS)
            def _():
                pltpu.async_copy(x_hbm.at[pl.ds(off + 2 * _C, _C)], in_v, in_s)

    for b in range(2):
        pltpu.make_async_copy(bufs[b][1], o_hbm.at[pl.ds(base, _C)], bufs[b][3]).wait()


def kernel(X):
    M, N = X.shape
    x = X.reshape(-1)
    mesh = plsc.VectorSubcoreMesh(core_axis_name="c", subcore_axis_name="s")
    out = pl.kernel(
        _sc_body,
        mesh=mesh,
        out_type=jax.ShapeDtypeStruct((_T,), jnp.float32),
        scratch_types=[
            pltpu.VMEM((_C,), jnp.float32),
            pltpu.VMEM((_C,), jnp.float32),
            pltpu.VMEM((_C,), jnp.float32),
            pltpu.VMEM((_C,), jnp.float32),
            pltpu.SemaphoreType.DMA,
            pltpu.SemaphoreType.DMA,
            pltpu.SemaphoreType.DMA,
            pltpu.SemaphoreType.DMA,
        ],
    )(x)
    return out.reshape(M, N)


# speedup vs baseline: 6.1157x; 3.1643x over previous
import jax
import jax.numpy as jnp
from jax import lax
from jax.experimental import pallas as pl
from jax.experimental.pallas import tpu as pltpu
from jax.experimental.pallas import tpu_sc as plsc

_ALPHA = 0.1

_NC, _NS, _L = 2, 16, 16
_NW = _NC * _NS
_M, _N = 16384, 4096
_ROWS_W = _M // _NW
_CR, _CC = 8, 2048
_COLS_PER_ROWBAND = _N // _CC
_CHUNKS = (_ROWS_W // _CR) * _COLS_PER_ROWBAND
_NB = 4
_P = 2


def _chunk_slice(x_hbm, base_row, k):
    r0 = base_row + (k // _COLS_PER_ROWBAND) * _CR
    c0 = (k % _COLS_PER_ROWBAND) * _CC
    return x_hbm.at[pl.ds(r0, _CR), pl.ds(c0, _CC)]


def _sc_body(x_hbm, o_hbm, b0, b1, b2, b3, si0, si1, si2, si3,
             so0, so1, so2, so3):
    wid = lax.axis_index("s") * _NC + lax.axis_index("c")
    base_row = wid * _ROWS_W
    bufs = (b0, b1, b2, b3)
    in_s = (si0, si1, si2, si3)
    out_s = (so0, so1, so2, so3)

    for k in range(_P):
        pltpu.async_copy(_chunk_slice(x_hbm, base_row, k), bufs[k], in_s[k])

    @pl.loop(0, _CHUNKS, step=_NB)
    def _outer(g):
        for b in range(_NB):
            k = g + b
            bn = (b + _P) % _NB
            @pl.when(k >= _NB - _P)
            def _():
                pltpu.make_async_copy(
                    bufs[bn], _chunk_slice(o_hbm, base_row, k), out_s[bn]
                ).wait()

            @pl.when(k + _P < _CHUNKS)
            def _():
                pltpu.async_copy(
                    _chunk_slice(x_hbm, base_row, k + _P), bufs[bn], in_s[bn]
                )

            pltpu.make_async_copy(
                _chunk_slice(x_hbm, base_row, k), bufs[b], in_s[b]
            ).wait()

            buf = bufs[b]
            for r in range(_CR):
                @plsc.parallel_loop(0, _CC, step=_L, unroll=8)
                def _compute(i):
                    v = buf[r, pl.ds(i, _L)]
                    buf[r, pl.ds(i, _L)] = v + jnp.where(v > -_ALPHA, -_ALPHA, _ALPHA)

            pltpu.async_copy(bufs[b], _chunk_slice(o_hbm, base_row, k), out_s[b])

    for k in range(_CHUNKS - (_NB - _P), _CHUNKS):
        b = k % _NB
        pltpu.make_async_copy(
            bufs[b], _chunk_slice(o_hbm, base_row, k), out_s[b]
        ).wait()


def kernel(X):
    mesh = plsc.VectorSubcoreMesh(core_axis_name="c", subcore_axis_name="s")
    return pl.kernel(
        _sc_body,
        mesh=mesh,
        out_type=jax.ShapeDtypeStruct((_M, _N), jnp.float32),
        scratch_types=(
            [pltpu.VMEM((_CR, _CC), jnp.float32)] * _NB
            + [pltpu.SemaphoreType.DMA] * (2 * _NB)
        ),
        compiler_params=pltpu.CompilerParams(use_tc_tiling_on_sc=True),
    )(X)
